# Initial kernel scaffold; baseline (speedup 1.0000x reference)
import jax
import jax.numpy as jnp
from jax.experimental import pallas as pl


def kernel(input_embedding, rel, relToCluIdx, rl_clu_embed, rl_rel_embed):
    raise NotImplementedError("write your pallas kernel here")


# speedup vs baseline: 2.9361x; 2.9361x over previous
import functools

import jax
import jax.numpy as jnp
from jax import lax
from jax.experimental import pallas as pl
from jax.experimental.pallas import tpu as pltpu
from jax.experimental.pallas import tpu_sc as plsc

REL_NUM = 256
CLUSTER = 64
DIM = 64
BATCH = 16384

NUM_CORES = 2
NUM_SUBCORES = 16
NUM_WORKERS = NUM_CORES * NUM_SUBCORES
ROWS_PER_W = BATCH // NUM_WORKERS
GROUPS = ROWS_PER_W // 16


def _sc_body(x_hbm, rel_hbm, map_hbm, clu_hbm, rele_hbm, out_hbm,
             x_v, idx_v, map_v, clu_v, rele_v, out_v, sem):
    wid = lax.axis_index("s") * NUM_CORES + lax.axis_index("c")
    base = wid * ROWS_PER_W

    x_copy = pltpu.async_copy(x_hbm.at[pl.ds(base * DIM, ROWS_PER_W * DIM)],
                              x_v, sem)
    pltpu.sync_copy(rel_hbm.at[pl.ds(base, ROWS_PER_W)], idx_v)
    pltpu.sync_copy(map_hbm, map_v)
    pltpu.sync_copy(rele_hbm, rele_v)
    pltpu.sync_copy(clu_hbm, clu_v)
    x_copy.wait()

    iota16 = lax.iota(jnp.int32, 16)

    def group_body(g, _):
        r16 = idx_v[pl.ds(g * 16, 16)]
        c16 = plsc.load_gather(map_v, [r16])
        rbase = r16 * DIM
        cbase = c16 * DIM
        xbase = (iota16 + g * 16) * DIM

        def d_body(d, acc):
            dv = jnp.full((16,), d, jnp.int32)
            tv = plsc.load_gather(rele_v, [rbase + dv])
            cv = plsc.load_gather(clu_v, [cbase + dv])
            xv = plsc.load_gather(x_v, [xbase + dv])
            return acc + (tv + cv) * xv

        acc = lax.fori_loop(0, DIM, d_body, jnp.zeros((16,), jnp.float32))
        out_v[pl.ds(g * 16, 16)] = acc
        return 0

    lax.fori_loop(0, GROUPS, group_body, 0)
    pltpu.sync_copy(out_v, out_hbm.at[pl.ds(base, ROWS_PER_W)])


def kernel(input_embedding, rel, relToCluIdx, rl_clu_embed, rl_rel_embed):
    x_flat = input_embedding.reshape(-1)
    clu_flat = rl_clu_embed.reshape(-1)
    rele_flat = rl_rel_embed.reshape(-1)

    mesh = plsc.VectorSubcoreMesh(core_axis_name="c", subcore_axis_name="s")
    run = functools.partial(
        pl.kernel,
        out_type=jax.ShapeDtypeStruct((BATCH,), jnp.float32),
        mesh=mesh,
        scratch_types=[
            pltpu.VMEM((ROWS_PER_W * DIM,), jnp.float32),
            pltpu.VMEM((ROWS_PER_W,), jnp.int32),
            pltpu.VMEM((REL_NUM,), jnp.int32),
            pltpu.VMEM((CLUSTER * DIM,), jnp.float32),
            pltpu.VMEM((REL_NUM * DIM,), jnp.float32),
            pltpu.VMEM((ROWS_PER_W,), jnp.float32),
            pltpu.SemaphoreType.DMA,
        ],
        compiler_params=pltpu.CompilerParams(needs_layout_passes=False),
    )(_sc_body)
    return run(x_flat, rel, relToCluIdx, clu_flat, rele_flat)


# speedup vs baseline: 3.3403x; 1.1376x over previous
import functools

import jax
import jax.numpy as jnp
from jax import lax
from jax.experimental import pallas as pl
from jax.experimental.pallas import tpu as pltpu
from jax.experimental.pallas import tpu_sc as plsc

REL_NUM = 256
CLUSTER = 64
DIM = 64
BATCH = 16384

NUM_CORES = 2
NUM_SUBCORES = 16
NUM_WORKERS = NUM_CORES * NUM_SUBCORES
ROWS_PER_W = BATCH // NUM_WORKERS
GROUPS = ROWS_PER_W // 16


def _sc_body(x_hbm, rel_hbm, map_hbm, clu_hbm, rele_hbm, out_hbm,
             x_v, idx_v, map_v, clu_v, rele_v, out_v, sem):
    wid = lax.axis_index("s") * NUM_CORES + lax.axis_index("c")
    base = wid * ROWS_PER_W

    x_copy = pltpu.async_copy(x_hbm.at[pl.ds(base * DIM, ROWS_PER_W * DIM)],
                              x_v, sem)
    pltpu.sync_copy(rel_hbm.at[pl.ds(base, ROWS_PER_W)], idx_v)
    pltpu.sync_copy(map_hbm, map_v)
    pltpu.sync_copy(rele_hbm, rele_v)
    pltpu.sync_copy(clu_hbm, clu_v)
    x_copy.wait()

    iota16 = lax.iota(jnp.int32, 16)

    def group_body(g, _):
        r16 = idx_v[pl.ds(g * 16, 16)]
        c16 = plsc.load_gather(map_v, [r16])
        rbase = r16 * DIM
        cbase = c16 * DIM
        xbase = (iota16 + g * 16) * DIM

        acc = jnp.zeros((16,), jnp.float32)
        for d in range(DIM):
            tv = plsc.load_gather(rele_v, [rbase + d])
            cv = plsc.load_gather(clu_v, [cbase + d])
            xv = plsc.load_gather(x_v, [xbase + d])
            acc = acc + (tv + cv) * xv
        out_v[pl.ds(g * 16, 16)] = acc
        return 0

    lax.fori_loop(0, GROUPS, group_body, 0)
    pltpu.sync_copy(out_v, out_hbm.at[pl.ds(base, ROWS_PER_W)])


def kernel(input_embedding, rel, relToCluIdx, rl_clu_embed, rl_rel_embed):
    x_flat = input_embedding.reshape(-1)
    clu_flat = rl_clu_embed.reshape(-1)
    rele_flat = rl_rel_embed.reshape(-1)

    mesh = plsc.VectorSubcoreMesh(core_axis_name="c", subcore_axis_name="s")
    run = functools.partial(
        pl.kernel,
        out_type=jax.ShapeDtypeStruct((BATCH,), jnp.float32),
        mesh=mesh,
        scratch_types=[
            pltpu.VMEM((ROWS_PER_W * DIM,), jnp.float32),
            pltpu.VMEM((ROWS_PER_W,), jnp.int32),
            pltpu.VMEM((REL_NUM,), jnp.int32),
            pltpu.VMEM((CLUSTER * DIM,), jnp.float32),
            pltpu.VMEM((REL_NUM * DIM,), jnp.float32),
            pltpu.VMEM((ROWS_PER_W,), jnp.float32),
            pltpu.SemaphoreType.DMA,
        ],
        compiler_params=pltpu.CompilerParams(needs_layout_passes=False),
    )(_sc_body)
    return run(x_flat, rel, relToCluIdx, clu_flat, rele_flat)


# speedup vs baseline: 4.3329x; 1.2972x over previous
import functools

import jax
import jax.numpy as jnp
from jax import lax
from jax.experimental import pallas as pl
from jax.experimental.pallas import tpu as pltpu
from jax.experimental.pallas import tpu_sc as plsc

REL_NUM = 256
CLUSTER = 64
DIM = 64
BATCH = 16384

NUM_CORES = 2
NUM_SUBCORES = 16
NUM_WORKERS = NUM_CORES * NUM_SUBCORES
ROWS_PER_W = BATCH // NUM_WORKERS
CHUNK = 128
NCHUNK = ROWS_PER_W // CHUNK
GROUPS = ROWS_PER_W // 16


def _sc_body(x_hbm, rel_hbm, map_hbm, clu_hbm, rele_hbm, out_hbm,
             x_v, rrows_v, crows_v, idx_v, cidx_v, map_v, out_v, sem, gsem):
    wid = lax.axis_index("s") * NUM_CORES + lax.axis_index("c")
    base = wid * ROWS_PER_W

    x_copy = pltpu.async_copy(x_hbm.at[pl.ds(base, ROWS_PER_W)], x_v, sem)
    pltpu.sync_copy(rel_hbm.at[pl.ds(wid * NCHUNK, NCHUNK)], idx_v)
    pltpu.sync_copy(map_hbm, map_v)

    rel_copies = [
        pltpu.async_copy(rele_hbm.at[idx_v.at[j]],
                         rrows_v.at[pl.ds(j * CHUNK, CHUNK)], gsem)
        for j in range(NCHUNK)
    ]

    for j in range(NCHUNK):
        for t in range(CHUNK // 16):
            r16 = idx_v[j, pl.ds(t * 16, 16)]
            cidx_v[j, pl.ds(t * 16, 16)] = plsc.load_gather(map_v, [r16])

    clu_copies = [
        pltpu.async_copy(clu_hbm.at[cidx_v.at[j]],
                         crows_v.at[pl.ds(j * CHUNK, CHUNK)], gsem)
        for j in range(NCHUNK)
    ]
    for c in rel_copies + clu_copies:
        c.wait()
    x_copy.wait()

    iota16 = lax.iota(jnp.int32, 16)

    def group_body(g, _):
        i0 = g * 16
        sums = jnp.zeros((16,), jnp.float32)
        for u in range(16):
            i = i0 + u
            acc = jnp.zeros((16,), jnp.float32)
            for k in range(0, DIM, 16):
                acc = acc + ((rrows_v[i, pl.ds(k, 16)]
                              + crows_v[i, pl.ds(k, 16)])
                             * x_v[i, pl.ds(k, 16)])
            sums = jnp.where(iota16 == u, jnp.sum(acc), sums)
        out_v[pl.ds(i0, 16)] = sums
        return 0

    lax.fori_loop(0, GROUPS, group_body, 0)
    pltpu.sync_copy(out_v, out_hbm.at[pl.ds(base, ROWS_PER_W)])


def kernel(input_embedding, rel, relToCluIdx, rl_clu_embed, rl_rel_embed):
    rel2 = rel.reshape(BATCH // CHUNK, CHUNK)

    mesh = plsc.VectorSubcoreMesh(core_axis_name="c", subcore_axis_name="s")
    run = functools.partial(
        pl.kernel,
        out_type=jax.ShapeDtypeStruct((BATCH,), jnp.float32),
        mesh=mesh,
        scratch_types=[
            pltpu.VMEM((ROWS_PER_W, DIM), jnp.float32),
            pltpu.VMEM((ROWS_PER_W, DIM), jnp.float32),
            pltpu.VMEM((ROWS_PER_W, DIM), jnp.float32),
            pltpu.VMEM((NCHUNK, CHUNK), jnp.int32),
            pltpu.VMEM((NCHUNK, CHUNK), jnp.int32),
            pltpu.VMEM((REL_NUM,), jnp.int32),
            pltpu.VMEM((ROWS_PER_W,), jnp.float32),
            pltpu.SemaphoreType.DMA,
            pltpu.SemaphoreType.DMA,
        ],
        compiler_params=pltpu.CompilerParams(needs_layout_passes=False,
                                             use_tc_tiling_on_sc=False),
    )(_sc_body)
    return run(input_embedding, rel2, relToCluIdx, rl_clu_embed, rl_rel_embed)


# speedup vs baseline: 4.4050x; 1.0166x over previous
import functools

import jax
import jax.numpy as jnp
from jax import lax
from jax.experimental import pallas as pl
from jax.experimental.pallas import tpu as pltpu
from jax.experimental.pallas import tpu_sc as plsc

REL_NUM = 256
CLUSTER = 64
DIM = 64
BATCH = 16384

NUM_CORES = 2
NUM_SUBCORES = 16
NUM_WORKERS = NUM_CORES * NUM_SUBCORES
ROWS_PER_W = BATCH // NUM_WORKERS
CHUNK = 128
NCHUNK = ROWS_PER_W // CHUNK
XCOLS = 128
XROWS_PER_W = ROWS_PER_W // 2
UNROLL = 8


def _sc_body(x_hbm, rel_hbm, map_hbm, clu_hbm, rele_hbm, out_hbm,
             x_v, rrows_v, crows_v, idx_v, cidx_v, map_v, out_v, sem, gsem):
    wid = lax.axis_index("s") * NUM_CORES + lax.axis_index("c")
    base = wid * ROWS_PER_W

    x_copy = pltpu.async_copy(x_hbm.at[pl.ds(wid * XROWS_PER_W, XROWS_PER_W)],
                              x_v, sem)
    pltpu.sync_copy(rel_hbm.at[pl.ds(wid * NCHUNK, NCHUNK)], idx_v)
    pltpu.sync_copy(map_hbm, map_v)

    rel_copies = [
        pltpu.async_copy(rele_hbm.at[idx_v.at[j]],
                         rrows_v.at[pl.ds(j * CHUNK, CHUNK)], gsem)
        for j in range(NCHUNK)
    ]

    for j in range(NCHUNK):
        for t in range(CHUNK // 16):
            r16 = idx_v[j, pl.ds(t * 16, 16)]
            cidx_v[j, pl.ds(t * 16, 16)] = plsc.load_gather(map_v, [r16])

    clu_copies = [
        pltpu.async_copy(clu_hbm.at[cidx_v.at[j]],
                         crows_v.at[pl.ds(j * CHUNK, CHUNK)], gsem)
        for j in range(NCHUNK)
    ]
    for c in rel_copies + clu_copies:
        c.wait()
    x_copy.wait()

    iota16 = lax.iota(jnp.int32, 16)
    lane15 = iota16 == 15

    def row_block(b, _):
        i0 = b * UNROLL
        for u in range(UNROLL):
            i = i0 + u
            acc = jnp.zeros((16,), jnp.float32)
            for k in range(0, DIM, 16):
                xrow = (i0 // 2) + (u // 2)
                xoff = (u % 2) * DIM + k
                acc = acc + ((rrows_v[i, pl.ds(k, 16)]
                              + crows_v[i, pl.ds(k, 16)])
                             * x_v[xrow, pl.ds(xoff, 16)])
            tot = plsc.cumsum(acc)
            plsc.store_scatter(out_v, [jnp.full((16,), i, jnp.int32)], tot,
                               mask=lane15)
        return 0

    lax.fori_loop(0, ROWS_PER_W // UNROLL, row_block, 0)
    pltpu.sync_copy(out_v, out_hbm.at[pl.ds(base, ROWS_PER_W)])


def kernel(input_embedding, rel, relToCluIdx, rl_clu_embed, rl_rel_embed):
    x2 = input_embedding.reshape(BATCH // 2, XCOLS)
    rel2 = rel.reshape(BATCH // CHUNK, CHUNK)

    mesh = plsc.VectorSubcoreMesh(core_axis_name="c", subcore_axis_name="s")
    run = functools.partial(
        pl.kernel,
        out_type=jax.ShapeDtypeStruct((BATCH,), jnp.float32),
        mesh=mesh,
        scratch_types=[
            pltpu.VMEM((XROWS_PER_W, XCOLS), jnp.float32),
            pltpu.VMEM((ROWS_PER_W, DIM), jnp.float32),
            pltpu.VMEM((ROWS_PER_W, DIM), jnp.float32),
            pltpu.VMEM((NCHUNK, CHUNK), jnp.int32),
            pltpu.VMEM((NCHUNK, CHUNK), jnp.int32),
            pltpu.VMEM((REL_NUM,), jnp.int32),
            pltpu.VMEM((ROWS_PER_W,), jnp.float32),
            pltpu.SemaphoreType.DMA,
            pltpu.SemaphoreType.DMA,
        ],
        compiler_params=pltpu.CompilerParams(needs_layout_passes=False,
                                             use_tc_tiling_on_sc=False),
    )(_sc_body)
    return run(x2, rel2, relToCluIdx, rl_clu_embed, rl_rel_embed)


# speedup vs baseline: 4.5750x; 1.0386x over previous
import functools

import jax
import jax.numpy as jnp
from jax import lax
from jax.experimental import pallas as pl
from jax.experimental.pallas import tpu as pltpu
from jax.experimental.pallas import tpu_sc as plsc

REL_NUM = 256
CLUSTER = 64
DIM = 64
BATCH = 16384

NUM_CORES = 2
NUM_SUBCORES = 16
NUM_WORKERS = NUM_CORES * NUM_SUBCORES
ROWS_PER_W = BATCH // NUM_WORKERS
CHUNK = 128
NCHUNK = ROWS_PER_W // CHUNK
XCOLS = 128
XROWS_PER_W = ROWS_PER_W // 2
UNROLL = 8

_PERM = [b + (j % 2) * 16 + j // 2 for b in (0, 32) for j in range(32)]


def _expand_bf16(pair_i32):
    lo = plsc.bitcast(lax.shift_left(pair_i32, 16), jnp.float32)
    hi = plsc.bitcast(jnp.bitwise_and(pair_i32, jnp.int32(-65536)),
                      jnp.float32)
    return lo, hi


def _sc_body(x_hbm, rel_hbm, map_hbm, clu_hbm, rele_hbm, out_hbm,
             x_v, rrows_v, crows_v, idx_v, cidx_v, map_v, out_v, sem, gsem):
    wid = lax.axis_index("s") * NUM_CORES + lax.axis_index("c")
    base = wid * ROWS_PER_W

    x_copy = pltpu.async_copy(x_hbm.at[pl.ds(wid * XROWS_PER_W, XROWS_PER_W)],
                              x_v, sem)
    pltpu.sync_copy(rel_hbm.at[pl.ds(wid * NCHUNK, NCHUNK)], idx_v)
    pltpu.sync_copy(map_hbm, map_v)

    rel_copies = [
        pltpu.async_copy(rele_hbm.at[idx_v.at[j]],
                         rrows_v.at[pl.ds(j * CHUNK, CHUNK)], gsem)
        for j in range(NCHUNK)
    ]

    for j in range(NCHUNK):
        for t in range(CHUNK // 16):
            r16 = idx_v[j, pl.ds(t * 16, 16)]
            cidx_v[j, pl.ds(t * 16, 16)] = plsc.load_gather(map_v, [r16])

    clu_copies = [
        pltpu.async_copy(clu_hbm.at[cidx_v.at[j]],
                         crows_v.at[pl.ds(j * CHUNK, CHUNK)], gsem)
        for j in range(NCHUNK)
    ]
    for c in rel_copies + clu_copies:
        c.wait()
    x_copy.wait()

    iota16 = lax.iota(jnp.int32, 16)
    lane15 = iota16 == 15

    def row_block(b, _):
        i0 = b * UNROLL
        ivec = jnp.full((16,), i0, jnp.int32)
        for u in range(UNROLL):
            i = i0 + u
            xrow = (i0 // 2) + (u // 2)
            xoff = (u % 2) * DIM
            acc0 = jnp.zeros((16,), jnp.float32)
            acc1 = jnp.zeros((16,), jnp.float32)
            for k in (0, 32):
                s32 = (rrows_v[i, pl.ds(k, 32)] + crows_v[i, pl.ds(k, 32)])
                lo, hi = _expand_bf16(plsc.bitcast(s32, jnp.int32))
                acc0 = acc0 + lo * x_v[xrow, pl.ds(xoff + k, 16)]
                acc1 = acc1 + hi * x_v[xrow, pl.ds(xoff + k + 16, 16)]
            tot = plsc.cumsum(acc0 + acc1)
            plsc.store_scatter(out_v, [ivec + u], tot, mask=lane15)
        return 0

    lax.fori_loop(0, ROWS_PER_W // UNROLL, row_block, 0)
    pltpu.sync_copy(out_v, out_hbm.at[pl.ds(base, ROWS_PER_W)])


def kernel(input_embedding, rel, relToCluIdx, rl_clu_embed, rl_rel_embed):
    x2 = input_embedding.reshape(BATCH // 2, XCOLS)
    rel2 = rel.reshape(BATCH // CHUNK, CHUNK)
    perm = jnp.asarray(_PERM, dtype=jnp.int32)
    clu_p = rl_clu_embed[:, perm].astype(jnp.bfloat16)
    rele_p = rl_rel_embed[:, perm].astype(jnp.bfloat16)

    mesh = plsc.VectorSubcoreMesh(core_axis_name="c", subcore_axis_name="s")
    run = functools.partial(
        pl.kernel,
        out_type=jax.ShapeDtypeStruct((BATCH,), jnp.float32),
        mesh=mesh,
        scratch_types=[
            pltpu.VMEM((XROWS_PER_W, XCOLS), jnp.float32),
            pltpu.VMEM((ROWS_PER_W, DIM), jnp.bfloat16),
            pltpu.VMEM((ROWS_PER_W, DIM), jnp.bfloat16),
            pltpu.VMEM((NCHUNK, CHUNK), jnp.int32),
            pltpu.VMEM((NCHUNK, CHUNK), jnp.int32),
            pltpu.VMEM((REL_NUM,), jnp.int32),
            pltpu.VMEM((ROWS_PER_W,), jnp.float32),
            pltpu.SemaphoreType.DMA,
            pltpu.SemaphoreType.DMA,
        ],
        compiler_params=pltpu.CompilerParams(needs_layout_passes=False,
                                             use_tc_tiling_on_sc=False),
    )(_sc_body)
    return run(x2, rel2, relToCluIdx, clu_p, rele_p)


# speedup vs baseline: 5.3100x; 1.1606x over previous
import functools

import jax
import jax.numpy as jnp
from jax import lax
from jax.experimental import pallas as pl
from jax.experimental.pallas import tpu as pltpu
from jax.experimental.pallas import tpu_sc as plsc

REL_NUM = 256
CLUSTER = 64
DIM = 64
BATCH = 16384

NUM_CORES = 2
NUM_SUBCORES = 16
NUM_WORKERS = NUM_CORES * NUM_SUBCORES
ROWS_PER_W = BATCH // NUM_WORKERS
CHUNK = 128
NCHUNK = ROWS_PER_W // CHUNK
XCOLS = DIM
XROWS_PER_W = ROWS_PER_W
UNROLL = 8

_PERM = [b + (j % 2) * 16 + j // 2 for b in (0, 32) for j in range(32)]


def _expand_bf16(pair_i32):
    lo = plsc.bitcast(lax.shift_left(pair_i32, 16), jnp.float32)
    hi = plsc.bitcast(jnp.bitwise_and(pair_i32, jnp.int32(-65536)),
                      jnp.float32)
    return lo, hi


def _sc_body(x_hbm, rel_hbm, map_hbm, clu_hbm, rele_hbm, out_hbm,
             x_v, rrows_v, crows_v, idx_v, cidx_v, map_v, out_v, sem, gsem):
    wid = lax.axis_index("s") * NUM_CORES + lax.axis_index("c")
    base = wid * ROWS_PER_W

    xh = XROWS_PER_W // 2
    x_copies = [
        pltpu.async_copy(x_hbm.at[pl.ds(wid * XROWS_PER_W + h * xh, xh)],
                         x_v.at[pl.ds(h * xh, xh)], sem)
        for h in range(2)
    ]
    pltpu.sync_copy(rel_hbm.at[pl.ds(wid * NCHUNK, NCHUNK)], idx_v)
    pltpu.sync_copy(map_hbm, map_v)

    rel_copies = [
        pltpu.async_copy(rele_hbm.at[idx_v.at[j]],
                         rrows_v.at[pl.ds(j * CHUNK, CHUNK)], gsem)
        for j in range(NCHUNK)
    ]

    for j in range(NCHUNK):
        for t in range(CHUNK // 16):
            r16 = idx_v[j, pl.ds(t * 16, 16)]
            cidx_v[j, pl.ds(t * 16, 16)] = plsc.load_gather(map_v, [r16])

    clu_copies = [
        pltpu.async_copy(clu_hbm.at[cidx_v.at[j]],
                         crows_v.at[pl.ds(j * CHUNK, CHUNK)], gsem)
        for j in range(NCHUNK)
    ]
    for c in rel_copies + clu_copies:
        c.wait()

    iota16 = lax.iota(jnp.int32, 16)
    lane15 = iota16 == 15

    def row(i):
        acc0 = jnp.zeros((16,), jnp.float32)
        acc1 = jnp.zeros((16,), jnp.float32)
        for k in (0, 32):
            s32 = (rrows_v[i, pl.ds(k, 32)] + crows_v[i, pl.ds(k, 32)])
            lo, hi = _expand_bf16(plsc.bitcast(s32, jnp.int32))
            acc0 = acc0 + lo * x_v[i, pl.ds(k, 16)]
            acc1 = acc1 + hi * x_v[i, pl.ds(k + 16, 16)]
        tot = plsc.cumsum(acc0 + acc1)
        plsc.store_scatter(out_v, [jnp.full((16,), i, jnp.int32)], tot,
                           mask=lane15)

    half_rows = ROWS_PER_W // 2
    x_copies[0].wait()
    plsc.parallel_loop(0, half_rows, unroll=UNROLL)(row)
    x_copies[1].wait()
    plsc.parallel_loop(half_rows, ROWS_PER_W, unroll=UNROLL)(row)
    pltpu.sync_copy(out_v, out_hbm.at[pl.ds(base, ROWS_PER_W)])


def kernel(input_embedding, rel, relToCluIdx, rl_clu_embed, rl_rel_embed):
    x2 = input_embedding
    rel2 = rel.reshape(BATCH // CHUNK, CHUNK)
    def permute(t):
        r = t.shape[0]
        return (t.reshape(r, 2, 2, 16).swapaxes(2, 3)
                .reshape(r, DIM).astype(jnp.bfloat16))

    clu_p = permute(rl_clu_embed)
    rele_p = permute(rl_rel_embed)

    mesh = plsc.VectorSubcoreMesh(core_axis_name="c", subcore_axis_name="s")
    run = functools.partial(
        pl.kernel,
        out_type=jax.ShapeDtypeStruct((BATCH,), jnp.float32),
        mesh=mesh,
        scratch_types=[
            pltpu.VMEM((XROWS_PER_W, XCOLS), jnp.float32),
            pltpu.VMEM((ROWS_PER_W, DIM), jnp.bfloat16),
            pltpu.VMEM((ROWS_PER_W, DIM), jnp.bfloat16),
            pltpu.VMEM((NCHUNK, CHUNK), jnp.int32),
            pltpu.VMEM((NCHUNK, CHUNK), jnp.int32),
            pltpu.VMEM((REL_NUM,), jnp.int32),
            pltpu.VMEM((ROWS_PER_W,), jnp.float32),
            pltpu.SemaphoreType.DMA,
            pltpu.SemaphoreType.DMA,
        ],
        compiler_params=pltpu.CompilerParams(needs_layout_passes=False,
                                             use_tc_tiling_on_sc=False),
    )(_sc_body)
    return run(x2, rel2, relToCluIdx, clu_p, rele_p)
